# baseline (speedup 1.0000x reference)
import jax
import jax.numpy as jnp
from jax import lax
from jax.experimental import pallas as pl
from jax.experimental.pallas import tpu as pltpu
from jax.experimental.pallas import tpu_sc as plsc

N = 10000
E = 320000
D = 128
NP = 10240
NC = 2
NS = 16
NW = NC * NS
EPT = NP
EP = NW * EPT
CH = 128
NCH = EPT // CH
CPR = 2 * NCH
A0 = 40
A1 = CPR - A0
NSP = 10112
ROWS_PER_TILE = NSP // NS


def _mm_body(x_ref, w_ref, o_ref):
    o_ref[...] = jnp.dot(x_ref[...], w_ref[...],
                         preferred_element_type=jnp.float32)


def _tc_matmul(xp, wext):
    nb = NP // 1024
    return pl.pallas_call(
        _mm_body,
        grid=(nb,),
        in_specs=[
            pl.BlockSpec((1024, D), lambda i: (i, 0)),
            pl.BlockSpec((D, 256), lambda i: (0, 0)),
        ],
        out_specs=pl.BlockSpec((1024, 256), lambda i: (i, 0)),
        out_shape=jax.ShapeDtypeStruct((NP, 256), jnp.float32),
    )(xp, wext)


def _norm_mm_body(acc_ref, dnm_ref, b_ref, w_ref, o_ref):
    feat = acc_ref[0] + acc_ref[1]
    denom = jnp.sum(dnm_ref[...], axis=0)[:, None]
    h = feat / (denom + 1e-16) + b_ref[...]
    h = jnp.maximum(h, 0.0)
    o_ref[...] = jnp.dot(h, w_ref[...], preferred_element_type=jnp.float32)


def _tc_norm_matmul(acc, dnm, b, wext):
    nb = NP // 1024
    return pl.pallas_call(
        _norm_mm_body,
        grid=(nb,),
        in_specs=[
            pl.BlockSpec((2, 1024, D), lambda i: (0, i, 0)),
            pl.BlockSpec((NW, 1024), lambda i: (0, i)),
            pl.BlockSpec((1, D), lambda i: (0, 0)),
            pl.BlockSpec((D, 256), lambda i: (0, 0)),
        ],
        out_specs=pl.BlockSpec((1024, 256), lambda i: (i, 0)),
        out_shape=jax.ShapeDtypeStruct((NP, 256), jnp.float32),
    )(acc, dnm, b, wext)


def _final_body(acc_ref, dnm_ref, b_ref, o_ref):
    feat = acc_ref[0] + acc_ref[1]
    denom = jnp.sum(dnm_ref[...], axis=0)[:, None]
    h = feat / (denom + 1e-16) + b_ref[...]
    m = jnp.max(h, axis=1, keepdims=True)
    lse = jnp.log(jnp.sum(jnp.exp(h - m), axis=1, keepdims=True))
    o_ref[...] = h - m - lse


def _tc_final(acc, dnm, b):
    nb = NP // 1024
    return pl.pallas_call(
        _final_body,
        grid=(nb,),
        in_specs=[
            pl.BlockSpec((2, 1024, D), lambda i: (0, i, 0)),
            pl.BlockSpec((NW, 1024), lambda i: (0, i)),
            pl.BlockSpec((1, D), lambda i: (0, 0)),
        ],
        out_specs=pl.BlockSpec((1024, D), lambda i: (i, 0)),
        out_shape=jax.ShapeDtypeStruct((NP, D), jnp.float32),
    )(acc, dnm, b)


def _sc_p1_body(srcf_hbm, dstf_hbm, asrc_hbm, adst_hbm,
                ex_hbm, dnm_hbm,
                asrc_v, adst_v, srcf_v, dstf_v, ex_v, dnm_v):
    c = lax.axis_index("c")
    s = lax.axis_index("s")
    w = c * NS + s

    pltpu.sync_copy(asrc_hbm, asrc_v)
    pltpu.sync_copy(adst_hbm, adst_v)
    pltpu.sync_copy(srcf_hbm.at[w], srcf_v)
    pltpu.sync_copy(dstf_hbm.at[w], dstf_v)

    zero16 = jnp.zeros((16,), jnp.float32)

    def z1(k, _):
        dnm_v[pl.ds(k * 16, 16)] = zero16
        return 0

    lax.fori_loop(0, NP // 16, z1, 0)

    lane = lax.iota(jnp.int32, 16)
    gbase = w * EPT

    def p1(k, _):
        sv = srcf_v[pl.ds(k * 16, 16)]
        dv = dstf_v[pl.ds(k * 16, 16)]
        e = (plsc.load_gather(asrc_v, [sv])
             + plsc.load_gather(adst_v, [dv]))
        e = jnp.maximum(e, e * 0.2)
        ex = jnp.exp(e)
        ex = jnp.where(gbase + k * 16 + lane < E, ex, 0.0)
        ex_v[pl.ds(k * 16, 16)] = ex
        plsc.addupdate_scatter(dnm_v, [dv], ex)
        return 0

    lax.fori_loop(0, EPT // 16, p1, 0)
    pltpu.sync_copy(ex_v, ex_hbm.at[w])
    pltpu.sync_copy(dnm_v, dnm_hbm.at[w])


def _sc_phase1(srcf, dstf, asrc, adst):
    mesh = plsc.VectorSubcoreMesh(core_axis_name="c", subcore_axis_name="s")
    f = pl.kernel(
        _sc_p1_body,
        out_type=(jax.ShapeDtypeStruct((NW, EPT), jnp.float32),
                  jax.ShapeDtypeStruct((NW, NP), jnp.float32)),
        mesh=mesh,
        compiler_params=pltpu.CompilerParams(needs_layout_passes=False),
        scratch_types=[
            pltpu.VMEM((NP,), jnp.float32),
            pltpu.VMEM((NP,), jnp.float32),
            pltpu.VMEM((EPT,), jnp.int32),
            pltpu.VMEM((EPT,), jnp.int32),
            pltpu.VMEM((EPT,), jnp.float32),
            pltpu.VMEM((NP,), jnp.float32),
        ],
    )
    return f(srcf, dstf, asrc, adst)


def _sc_p2_body(src4_hbm, dst4_hbm, ex4_hbm, h_hbm, acc_hbm,
                dst3_v, srcc, exc, buf, spm_acc, isem, esem, gsem, ssem):
    c = lax.axis_index("c")
    s = lax.axis_index("s")

    zero16 = jnp.zeros((16,), jnp.float32)

    def pz(i, _):
        buf[0][i // (D // 16), pl.ds((i % (D // 16)) * 16, 16)] = zero16
        return 0

    lax.fori_loop(0, CH * (D // 16), pz, 0)
    base = s * ROWS_PER_TILE
    for off in range(0, ROWS_PER_TILE, CH):
        cnt = min(CH, ROWS_PER_TILE - off)
        pltpu.sync_copy(buf[0].at[pl.ds(0, cnt)],
                        spm_acc.at[pl.ds(base + off, cnt)])
    plsc.subcore_barrier()

    def pipeline(nch, kb):
        pltpu.sync_copy(dst4_hbm.at[s, pl.ds(kb, nch)],
                        dst3_v.at[pl.ds(0, nch)])

        def fetch_idx(k, j4):
            pltpu.async_copy(src4_hbm.at[s, kb + k], srcc[j4], isem[j4])
            pltpu.async_copy(ex4_hbm.at[s, kb + k], exc[j4], esem[j4])

        for k in range(4):
            fetch_idx(k, k)

        def issue(k, b2, j4, q, need_guard):
            def start():
                pltpu.make_async_copy(src4_hbm.at[s, kb], srcc[j4],
                                      isem[j4]).wait()
                pltpu.async_copy(h_hbm.at[srcc[j4]], buf[b2], gsem[b2])

            if need_guard:
                @pl.when(q >= 1)
                def _():
                    pltpu.make_async_copy(buf[b2],
                                          spm_acc.at[dst3_v.at[0]],
                                          ssem[b2]).wait()
                    start()

                @pl.when(q < 1)
                def _():
                    start()
            else:
                pltpu.make_async_copy(buf[b2], spm_acc.at[dst3_v.at[0]],
                                      ssem[b2]).wait()
                start()

        def complete(k, b2, j4, q, fetch_guarded):
            pltpu.make_async_copy(h_hbm.at[srcc[j4]], buf[b2],
                                  gsem[b2]).wait()
            pltpu.make_async_copy(ex4_hbm.at[s, kb], exc[j4],
                                  esem[j4]).wait()

            def scale(i, _):
                av = plsc.load_gather(exc[j4],
                                      [jnp.full((16,), i, jnp.int32)])
                for j in range(D // 16):
                    buf[b2][i, pl.ds(j * 16, 16)] = (
                        buf[b2][i, pl.ds(j * 16, 16)] * av)
                return 0

            lax.fori_loop(0, CH, scale, 0)
            pltpu.async_copy(buf[b2], spm_acc.at[dst3_v.at[k]], ssem[b2],
                             add=True)
            if fetch_guarded:
                @pl.when(q <= nch // 4 - 2)
                def _():
                    fetch_idx(k + 4, j4)
            else:
                fetch_idx(k + 4, j4)

        def p2_body(q, _):
            k0 = 4 * q
            issue(k0, 0, 0, q, need_guard=True)
            @pl.when(q >= 1)
            def _():
                complete(k0 - 1, 1, 3, q, fetch_guarded=False)
            issue(k0 + 1, 1, 1, q, need_guard=True)
            complete(k0, 0, 0, q, fetch_guarded=True)
            issue(k0 + 2, 0, 2, q, need_guard=False)
            complete(k0 + 1, 1, 1, q, fetch_guarded=True)
            issue(k0 + 3, 1, 3, q, need_guard=False)
            complete(k0 + 2, 0, 2, q, fetch_guarded=True)
            return 0

        lax.fori_loop(0, nch // 4, p2_body, 0)
        complete(nch - 1, 1, 3, nch // 4, fetch_guarded=True)
        pltpu.make_async_copy(buf[0], spm_acc.at[dst3_v.at[0]],
                              ssem[0]).wait()
        pltpu.make_async_copy(buf[1], spm_acc.at[dst3_v.at[0]],
                              ssem[1]).wait()

    @pl.when(c == 0)
    def _():
        pipeline(A0, 0)

    @pl.when(c == 1)
    def _():
        pipeline(A1, A0)

    plsc.subcore_barrier()

    pltpu.sync_copy(spm_acc.at[pl.ds(base, ROWS_PER_TILE)],
                    acc_hbm.at[c, pl.ds(base, ROWS_PER_TILE)])


def _sc_phase2(src4, dst4, ex4, h):
    mesh = plsc.VectorSubcoreMesh(core_axis_name="c", subcore_axis_name="s")
    f = pl.kernel(
        _sc_p2_body,
        out_type=jax.ShapeDtypeStruct((NC, NP, D), jnp.float32),
        mesh=mesh,
        compiler_params=pltpu.CompilerParams(needs_layout_passes=False),
        scratch_types=[
            pltpu.VMEM((max(A0, A1), CH), jnp.int32),
            [pltpu.VMEM((CH,), jnp.int32) for _ in range(4)],
            [pltpu.VMEM((CH,), jnp.float32) for _ in range(4)],
            [pltpu.VMEM((CH, D), jnp.float32) for _ in range(2)],
            pltpu.VMEM_SHARED((NSP, D), jnp.float32),
            [pltpu.SemaphoreType.DMA for _ in range(4)],
            [pltpu.SemaphoreType.DMA for _ in range(4)],
            [pltpu.SemaphoreType.DMA for _ in range(2)],
            [pltpu.SemaphoreType.DMA for _ in range(2)],
        ],
    )
    return f(src4, dst4, ex4, h)


def kernel(x, edge_index, W1, att_src1, att_dst1, b1,
           W2, att_src2, att_dst2, b2):
    src = jnp.pad(edge_index[0].astype(jnp.int32), (0, EP - E))
    dst = jnp.pad(edge_index[1].astype(jnp.int32), (0, EP - E))
    srcf = src.reshape(NW, EPT)
    dstf = dst.reshape(NW, EPT)
    src4 = src.reshape(NS, CPR, CH)
    dst4 = dst.reshape(NS, CPR, CH)

    def wext(W, a_s, a_d):
        return jnp.concatenate(
            [W, (W @ a_s)[:, None], (W @ a_d)[:, None],
             jnp.zeros((D, 256 - D - 2), jnp.float32)], axis=1)

    xp = jnp.pad(x, ((0, NP - N), (0, 0)))
    hext1 = _tc_matmul(xp, wext(W1, att_src1, att_dst1))
    ex1, dnm1 = _sc_phase1(srcf, dstf, hext1[:, D], hext1[:, D + 1])
    acc1 = _sc_phase2(src4, dst4, ex1.reshape(NS, CPR, CH), hext1[:, :D])
    hext2 = _tc_norm_matmul(acc1, dnm1, b1[None, :],
                            wext(W2, att_src2, att_dst2))
    ex2, dnm2 = _sc_phase1(srcf, dstf, hext2[:, D], hext2[:, D + 1])
    acc2 = _sc_phase2(src4, dst4, ex2.reshape(NS, CPR, CH), hext2[:, :D])
    out = _tc_final(acc2, dnm2, b2[None, :])
    return out[:N]


# speedup vs baseline: 1.3750x; 1.3750x over previous
import jax
import jax.numpy as jnp
from jax import lax
from jax.experimental import pallas as pl
from jax.experimental.pallas import tpu as pltpu
from jax.experimental.pallas import tpu_sc as plsc

N = 10000
E = 320000
D = 128
NP = 10240
NC = 2
NS = 16
NW = NC * NS
EPT = NP
EP = NW * EPT
CH = 128
NCH = EPT // CH
CPR = 2 * NCH
A0 = 120
A1 = CPR - A0
NSP = 10112
ROWS_PER_TILE = NSP // NS


def _mm_body(x_ref, w_ref, o_ref):
    o_ref[...] = jnp.dot(x_ref[...], w_ref[...],
                         preferred_element_type=jnp.float32)


def _tc_matmul(xp, wext):
    nb = NP // 1024
    return pl.pallas_call(
        _mm_body,
        grid=(nb,),
        in_specs=[
            pl.BlockSpec((1024, D), lambda i: (i, 0)),
            pl.BlockSpec((D, 256), lambda i: (0, 0)),
        ],
        out_specs=pl.BlockSpec((1024, 256), lambda i: (i, 0)),
        out_shape=jax.ShapeDtypeStruct((NP, 256), jnp.float32),
    )(xp, wext)


def _norm_mm_body(acc_ref, dnm_ref, b_ref, w_ref, o_ref):
    feat = acc_ref[0] + acc_ref[1]
    denom = jnp.sum(dnm_ref[...], axis=0)[:, None]
    h = feat / (denom + 1e-16) + b_ref[...]
    h = jnp.maximum(h, 0.0)
    o_ref[...] = jnp.dot(h, w_ref[...], preferred_element_type=jnp.float32)


def _tc_norm_matmul(acc, dnm, b, wext):
    nb = NP // 1024
    return pl.pallas_call(
        _norm_mm_body,
        grid=(nb,),
        in_specs=[
            pl.BlockSpec((2, 1024, D), lambda i: (0, i, 0)),
            pl.BlockSpec((NW, 1024), lambda i: (0, i)),
            pl.BlockSpec((1, D), lambda i: (0, 0)),
            pl.BlockSpec((D, 256), lambda i: (0, 0)),
        ],
        out_specs=pl.BlockSpec((1024, 256), lambda i: (i, 0)),
        out_shape=jax.ShapeDtypeStruct((NP, 256), jnp.float32),
    )(acc, dnm, b, wext)


def _final_body(acc_ref, dnm_ref, b_ref, o_ref):
    feat = acc_ref[0] + acc_ref[1]
    denom = jnp.sum(dnm_ref[...], axis=0)[:, None]
    h = feat / (denom + 1e-16) + b_ref[...]
    m = jnp.max(h, axis=1, keepdims=True)
    lse = jnp.log(jnp.sum(jnp.exp(h - m), axis=1, keepdims=True))
    o_ref[...] = h - m - lse


def _tc_final(acc, dnm, b):
    nb = NP // 1024
    return pl.pallas_call(
        _final_body,
        grid=(nb,),
        in_specs=[
            pl.BlockSpec((2, 1024, D), lambda i: (0, i, 0)),
            pl.BlockSpec((NW, 1024), lambda i: (0, i)),
            pl.BlockSpec((1, D), lambda i: (0, 0)),
        ],
        out_specs=pl.BlockSpec((1024, D), lambda i: (i, 0)),
        out_shape=jax.ShapeDtypeStruct((NP, D), jnp.float32),
    )(acc, dnm, b)


def _sc_p1_body(srcf_hbm, dstf_hbm, asrc_hbm, adst_hbm,
                ex_hbm, dnm_hbm,
                asrc_v, adst_v, srcf_v, dstf_v, ex_v, dnm_v):
    c = lax.axis_index("c")
    s = lax.axis_index("s")
    w = c * NS + s

    pltpu.sync_copy(asrc_hbm, asrc_v)
    pltpu.sync_copy(adst_hbm, adst_v)
    pltpu.sync_copy(srcf_hbm.at[w], srcf_v)
    pltpu.sync_copy(dstf_hbm.at[w], dstf_v)

    zero16 = jnp.zeros((16,), jnp.float32)

    def z1(k, _):
        dnm_v[pl.ds(k * 16, 16)] = zero16
        return 0

    lax.fori_loop(0, NP // 16, z1, 0)

    lane = lax.iota(jnp.int32, 16)
    gbase = w * EPT

    def p1(k, _):
        sv = srcf_v[pl.ds(k * 16, 16)]
        dv = dstf_v[pl.ds(k * 16, 16)]
        e = (plsc.load_gather(asrc_v, [sv])
             + plsc.load_gather(adst_v, [dv]))
        e = jnp.maximum(e, e * 0.2)
        ex = jnp.exp(e)
        ex = jnp.where(gbase + k * 16 + lane < E, ex, 0.0)
        ex_v[pl.ds(k * 16, 16)] = ex
        plsc.addupdate_scatter(dnm_v, [dv], ex)
        return 0

    lax.fori_loop(0, EPT // 16, p1, 0)
    pltpu.sync_copy(ex_v, ex_hbm.at[w])
    pltpu.sync_copy(dnm_v, dnm_hbm.at[w])


def _sc_phase1(srcf, dstf, asrc, adst):
    mesh = plsc.VectorSubcoreMesh(core_axis_name="c", subcore_axis_name="s")
    f = pl.kernel(
        _sc_p1_body,
        out_type=(jax.ShapeDtypeStruct((NW, EPT), jnp.float32),
                  jax.ShapeDtypeStruct((NW, NP), jnp.float32)),
        mesh=mesh,
        compiler_params=pltpu.CompilerParams(needs_layout_passes=False),
        scratch_types=[
            pltpu.VMEM((NP,), jnp.float32),
            pltpu.VMEM((NP,), jnp.float32),
            pltpu.VMEM((EPT,), jnp.int32),
            pltpu.VMEM((EPT,), jnp.int32),
            pltpu.VMEM((EPT,), jnp.float32),
            pltpu.VMEM((NP,), jnp.float32),
        ],
    )
    return f(srcf, dstf, asrc, adst)


def _sc_p2_body(src4_hbm, dst4_hbm, ex4_hbm, h_hbm, acc_hbm,
                dst3_v, srcc, exc, buf, spm_acc, isem, esem, gsem, ssem):
    c = lax.axis_index("c")
    s = lax.axis_index("s")

    zero16 = jnp.zeros((16,), jnp.float32)

    def pz(i, _):
        buf[0][i // (D // 16), pl.ds((i % (D // 16)) * 16, 16)] = zero16
        return 0

    lax.fori_loop(0, CH * (D // 16), pz, 0)
    base = s * ROWS_PER_TILE
    for off in range(0, ROWS_PER_TILE, CH):
        cnt = min(CH, ROWS_PER_TILE - off)
        pltpu.sync_copy(buf[0].at[pl.ds(0, cnt)],
                        spm_acc.at[pl.ds(base + off, cnt)])
    plsc.subcore_barrier()

    def pipeline(nch, kb):
        pltpu.sync_copy(dst4_hbm.at[s, pl.ds(kb, nch)],
                        dst3_v.at[pl.ds(0, nch)])

        def fetch_idx(k, j4):
            pltpu.async_copy(src4_hbm.at[s, kb + k], srcc[j4], isem[j4])
            pltpu.async_copy(ex4_hbm.at[s, kb + k], exc[j4], esem[j4])

        for k in range(4):
            fetch_idx(k, k)

        def issue(k, b2, j4, q, need_guard):
            def start():
                pltpu.make_async_copy(src4_hbm.at[s, kb], srcc[j4],
                                      isem[j4]).wait()
                pltpu.async_copy(h_hbm.at[srcc[j4]], buf[b2], gsem[b2])

            if need_guard:
                @pl.when(q >= 1)
                def _():
                    pltpu.make_async_copy(buf[b2],
                                          spm_acc.at[dst3_v.at[0]],
                                          ssem[b2]).wait()
                    start()

                @pl.when(q < 1)
                def _():
                    start()
            else:
                pltpu.make_async_copy(buf[b2], spm_acc.at[dst3_v.at[0]],
                                      ssem[b2]).wait()
                start()

        def complete(k, b2, j4, q, fetch_guarded):
            pltpu.make_async_copy(h_hbm.at[srcc[j4]], buf[b2],
                                  gsem[b2]).wait()
            pltpu.make_async_copy(ex4_hbm.at[s, kb], exc[j4],
                                  esem[j4]).wait()

            def scale(i, _):
                av = plsc.load_gather(exc[j4],
                                      [jnp.full((16,), i, jnp.int32)])
                for j in range(D // 16):
                    buf[b2][i, pl.ds(j * 16, 16)] = (
                        buf[b2][i, pl.ds(j * 16, 16)] * av)
                return 0

            lax.fori_loop(0, CH, scale, 0)
            pltpu.async_copy(buf[b2], spm_acc.at[dst3_v.at[k]], ssem[b2],
                             add=True)
            if fetch_guarded:
                @pl.when(q <= nch // 4 - 2)
                def _():
                    fetch_idx(k + 4, j4)
            else:
                fetch_idx(k + 4, j4)

        def p2_body(q, _):
            k0 = 4 * q
            issue(k0, 0, 0, q, need_guard=True)
            @pl.when(q >= 1)
            def _():
                complete(k0 - 1, 1, 3, q, fetch_guarded=False)
            issue(k0 + 1, 1, 1, q, need_guard=True)
            complete(k0, 0, 0, q, fetch_guarded=True)
            issue(k0 + 2, 0, 2, q, need_guard=False)
            complete(k0 + 1, 1, 1, q, fetch_guarded=True)
            issue(k0 + 3, 1, 3, q, need_guard=False)
            complete(k0 + 2, 0, 2, q, fetch_guarded=True)
            return 0

        lax.fori_loop(0, nch // 4, p2_body, 0)
        complete(nch - 1, 1, 3, nch // 4, fetch_guarded=True)
        pltpu.make_async_copy(buf[0], spm_acc.at[dst3_v.at[0]],
                              ssem[0]).wait()
        pltpu.make_async_copy(buf[1], spm_acc.at[dst3_v.at[0]],
                              ssem[1]).wait()

    @pl.when(c == 0)
    def _():
        pipeline(A0, 0)

    @pl.when(c == 1)
    def _():
        pipeline(A1, A0)

    plsc.subcore_barrier()

    pltpu.sync_copy(spm_acc.at[pl.ds(base, ROWS_PER_TILE)],
                    acc_hbm.at[c, pl.ds(base, ROWS_PER_TILE)])


def _sc_phase2(src4, dst4, ex4, h):
    mesh = plsc.VectorSubcoreMesh(core_axis_name="c", subcore_axis_name="s")
    f = pl.kernel(
        _sc_p2_body,
        out_type=jax.ShapeDtypeStruct((NC, NP, D), jnp.float32),
        mesh=mesh,
        compiler_params=pltpu.CompilerParams(needs_layout_passes=False),
        scratch_types=[
            pltpu.VMEM((max(A0, A1), CH), jnp.int32),
            [pltpu.VMEM((CH,), jnp.int32) for _ in range(4)],
            [pltpu.VMEM((CH,), jnp.float32) for _ in range(4)],
            [pltpu.VMEM((CH, D), jnp.float32) for _ in range(2)],
            pltpu.VMEM_SHARED((NSP, D), jnp.float32),
            [pltpu.SemaphoreType.DMA for _ in range(4)],
            [pltpu.SemaphoreType.DMA for _ in range(4)],
            [pltpu.SemaphoreType.DMA for _ in range(2)],
            [pltpu.SemaphoreType.DMA for _ in range(2)],
        ],
    )
    return f(src4, dst4, ex4, h)


def kernel(x, edge_index, W1, att_src1, att_dst1, b1,
           W2, att_src2, att_dst2, b2):
    src = jnp.pad(edge_index[0].astype(jnp.int32), (0, EP - E))
    dst = jnp.pad(edge_index[1].astype(jnp.int32), (0, EP - E))
    srcf = src.reshape(NW, EPT)
    dstf = dst.reshape(NW, EPT)
    src4 = src.reshape(NS, CPR, CH)
    dst4 = dst.reshape(NS, CPR, CH)

    def wext(W, a_s, a_d):
        return jnp.concatenate(
            [W, (W @ a_s)[:, None], (W @ a_d)[:, None],
             jnp.zeros((D, 256 - D - 2), jnp.float32)], axis=1)

    xp = jnp.pad(x, ((0, NP - N), (0, 0)))
    hext1 = _tc_matmul(xp, wext(W1, att_src1, att_dst1))
    ex1, dnm1 = _sc_phase1(srcf, dstf, hext1[:, D], hext1[:, D + 1])
    acc1 = _sc_phase2(src4, dst4, ex1.reshape(NS, CPR, CH), hext1[:, :D])
    hext2 = _tc_norm_matmul(acc1, dnm1, b1[None, :],
                            wext(W2, att_src2, att_dst2))
    ex2, dnm2 = _sc_phase1(srcf, dstf, hext2[:, D], hext2[:, D + 1])
    acc2 = _sc_phase2(src4, dst4, ex2.reshape(NS, CPR, CH), hext2[:, :D])
    out = _tc_final(acc2, dnm2, b2[None, :])
    return out[:N]
